# baseline (speedup 1.0000x reference)
import functools

import jax
import jax.numpy as jnp
from jax import lax
from jax.experimental import pallas as pl
from jax.experimental.pallas import tpu as pltpu
from jax.experimental.pallas import tpu_sc as plsc

_B = 16384
_D = 16
_NC = 2
_NS = 16
_NW = _NC * _NS
_RPW = 2 * _B // _NW
_BK = 16
_NBK = _RPW // _BK
_FL = 128


def _gather_body(ut, it, xT, u_out, i_out,
                 idx_v, bufa, bufb, bufc, stage_v, sema, semb, semc):
    wid = lax.axis_index("s") * _NC + lax.axis_index("c")
    tid = wid // (_NW // 2)
    base = (wid % (_NW // 2)) * _RPW
    iota16 = lax.iota(jnp.int32, 16)

    def one_table(tab, row, out):
        pltpu.sync_copy(xT.at[:, pl.ds(base, _RPW)], idx_v)

        def fire(b, buf, sem):
            vv = idx_v[row, pl.ds(b * _BK, _BK)]
            for s in range(_BK):
                r = vv[s]
                l = pl.multiple_of((r >> 7) << 7, 128)
                pltpu.async_copy(tab.at[:, pl.ds(l, 128)], buf.at[s], sem)

        def drain(buf, sem):
            for s in range(_BK):
                pltpu.make_async_copy(
                    tab.at[:, pl.ds(0, 128)], buf.at[s], sem).wait()

        def extract(b, buf, slot):
            vlo = idx_v[row, pl.ds(b * _BK, _BK)] & 127
            for f in range(_D):
                fv = jnp.full((_BK,), f, jnp.int32)
                vals = plsc.load_gather(buf, [iota16, fv, vlo])
                stage_v[f, pl.ds(slot, _BK)] = vals

        fire(0, bufa, sema)
        fire(1, bufb, semb)

        bufs = (bufa, bufb, bufc)
        sems = (sema, semb, semc)

        def step(b, buf, sem, nbuf, nsem):
            @pl.when(b + 2 < _NBK)
            def _():
                fire(b + 2, nbuf, nsem)

            drain(buf, sem)
            extract(b, buf, (b % (_FL // _BK)) * _BK)

            @pl.when((b + 1) % (_FL // _BK) == 0)
            def _():
                fbase = pl.multiple_of(
                    base + (b + 1 - _FL // _BK) * _BK, 128)
                pltpu.sync_copy(stage_v, out.at[:, pl.ds(fbase, _FL)])

        def triple(j, _):
            b0 = 3 * j
            for t in range(3):
                step(b0 + t, bufs[t], sems[t], bufs[(t + 2) % 3],
                     sems[(t + 2) % 3])
            return 0

        lax.fori_loop(0, _NBK // 3, triple, 0)
        step(_NBK - 1, bufs[(_NBK - 1) % 3], sems[(_NBK - 1) % 3],
             bufa, sema)

    @pl.when(tid == 0)
    def _():
        one_table(ut, 0, u_out)

    @pl.when(tid == 1)
    def _():
        one_table(it, 1, i_out)


_gather2 = functools.partial(
    pl.kernel,
    mesh=plsc.VectorSubcoreMesh(core_axis_name="c", subcore_axis_name="s"),
    out_type=(jax.ShapeDtypeStruct((_D, _B), jnp.float32),
              jax.ShapeDtypeStruct((_D, _B), jnp.float32)),
    scratch_types=[
        pltpu.VMEM((2, _RPW), jnp.int32),
        pltpu.VMEM((_BK, _D, 128), jnp.float32),
        pltpu.VMEM((_BK, _D, 128), jnp.float32),
        pltpu.VMEM((_BK, _D, 128), jnp.float32),
        pltpu.VMEM((_D, _FL), jnp.float32),
        pltpu.SemaphoreType.DMA,
        pltpu.SemaphoreType.DMA,
        pltpu.SemaphoreType.DMA,
    ],
    compiler_params=pltpu.CompilerParams(needs_layout_passes=False),
)(_gather_body)


def _mlp_body(uT_ref, iT_ref, w1uT_ref, w1iT_ref, b1_ref, w2T_ref, outT_ref):
    h = (w1uT_ref[...] @ uT_ref[...] + w1iT_ref[...] @ iT_ref[...]
         + b1_ref[...])
    outT_ref[...] = w2T_ref[...] @ jnp.maximum(h, 0.0)


_mlp = pl.pallas_call(
    _mlp_body,
    out_shape=jax.ShapeDtypeStruct((1, _B), jnp.float32),
)


def kernel(x, user_table, item_table, W1, b1, W2):
    uT, iT = _gather2(user_table.T, item_table.T, x.T)
    outT = _mlp(uT, iT, W1[:_D].T, W1[_D:].T, b1.reshape(_D, 1), W2.T)
    return (outT.T, uT.T, iT.T)
